# baseline (speedup 1.0000x reference)
import jax
import jax.numpy as jnp
from jax import lax
from jax.experimental import pallas as pl
from jax.experimental.pallas import tpu as pltpu

SEQ_LEN = 200
D_MODEL = 128
BATCH = 4096
BB = 32
NB = BATCH // BB
NSEM = 8


def _bcast_kernel(pos_ref, idx_ref, out_ref, scratch, sems):
    posb = jnp.broadcast_to(idx_ref[...], (SEQ_LEN, SEQ_LEN))
    onehot_t = (
        lax.broadcasted_iota(jnp.int32, (SEQ_LEN, SEQ_LEN), 0) == posb
    ).astype(jnp.float32)
    emb = lax.dot_general(
        onehot_t,
        pos_ref[...],
        dimension_numbers=(((0,), (0,)), ((), ())),
        preferred_element_type=jnp.float32,
    )
    MINI = 8
    scratch[0:MINI] = jnp.broadcast_to(emb[None], (MINI, SEQ_LEN, D_MODEL))
    for j in range(BB // MINI):
        pltpu.make_async_copy(
            scratch.at[pl.ds(0, MINI)],
            out_ref.at[pl.ds(j * MINI, MINI)],
            sems.at[j % NSEM],
        ).start()
    scratch[MINI:BB] = jnp.broadcast_to(emb[None], (BB - MINI, SEQ_LEN, D_MODEL))

    def _start(k, c):
        pltpu.make_async_copy(
            scratch, out_ref.at[pl.ds(k * BB, BB)], sems.at[k % NSEM]
        ).start()
        return c

    lax.fori_loop(1, NB, _start, None)

    for j in range(BB // MINI):
        pltpu.make_async_copy(
            scratch.at[pl.ds(0, MINI)],
            out_ref.at[pl.ds(j * MINI, MINI)],
            sems.at[j % NSEM],
        ).wait()

    def _wait(k, c):
        pltpu.make_async_copy(
            scratch, out_ref.at[pl.ds(k * BB, BB)], sems.at[k % NSEM]
        ).wait()
        return c

    lax.fori_loop(1, NB, _wait, None)


@jax.jit
def _run(pos_embed, positions):
    idx2d = positions.astype(jnp.int32).reshape(1, SEQ_LEN)
    return pl.pallas_call(
        _bcast_kernel,
        grid=(1,),
        in_specs=[
            pl.BlockSpec((SEQ_LEN, D_MODEL), lambda i: (0, 0)),
            pl.BlockSpec((1, SEQ_LEN), lambda i: (0, 0)),
        ],
        out_specs=pl.BlockSpec(memory_space=pl.ANY),
        out_shape=jax.ShapeDtypeStruct((BATCH, SEQ_LEN, D_MODEL), jnp.float32),
        scratch_shapes=[
            pltpu.VMEM((BB, SEQ_LEN, D_MODEL), jnp.float32),
            pltpu.SemaphoreType.DMA((NSEM,)),
        ],
        compiler_params=pltpu.CompilerParams(
            dimension_semantics=("arbitrary",),
        ),
    )(pos_embed, idx2d)


def kernel(batch_size, pos_embed, positions):
    return _run(pos_embed, positions)


# speedup vs baseline: 1.0069x; 1.0069x over previous
import jax
import jax.numpy as jnp
from jax import lax
from jax.experimental import pallas as pl
from jax.experimental.pallas import tpu as pltpu

SEQ_LEN = 200
D_MODEL = 128
BATCH = 4096
BB = 32
NB = BATCH // BB
NSEM = 16


def _bcast_kernel(pos_ref, idx_ref, out_ref, scratch, sems):
    posb = jnp.broadcast_to(idx_ref[...], (SEQ_LEN, SEQ_LEN))
    onehot_t = (
        lax.broadcasted_iota(jnp.int32, (SEQ_LEN, SEQ_LEN), 0) == posb
    ).astype(jnp.float32)
    emb = lax.dot_general(
        onehot_t,
        pos_ref[...],
        dimension_numbers=(((0,), (0,)), ((), ())),
        preferred_element_type=jnp.float32,
    )
    scratch[...] = jnp.broadcast_to(emb[None], (BB, SEQ_LEN, D_MODEL))

    def _start(k, c):
        pltpu.make_async_copy(
            scratch, out_ref.at[pl.ds(k * BB, BB)], sems.at[k % NSEM]
        ).start()
        return c

    lax.fori_loop(0, NB, _start, None)

    def _wait(k, c):
        pltpu.make_async_copy(
            scratch, out_ref.at[pl.ds(k * BB, BB)], sems.at[k % NSEM]
        ).wait()
        return c

    lax.fori_loop(0, NB, _wait, None)


@jax.jit
def _run(pos_embed, positions):
    idx2d = positions.astype(jnp.int32).reshape(1, SEQ_LEN)
    return pl.pallas_call(
        _bcast_kernel,
        grid=(1,),
        in_specs=[
            pl.BlockSpec((SEQ_LEN, D_MODEL), lambda i: (0, 0)),
            pl.BlockSpec((1, SEQ_LEN), lambda i: (0, 0)),
        ],
        out_specs=pl.BlockSpec(memory_space=pl.ANY),
        out_shape=jax.ShapeDtypeStruct((BATCH, SEQ_LEN, D_MODEL), jnp.float32),
        scratch_shapes=[
            pltpu.VMEM((BB, SEQ_LEN, D_MODEL), jnp.float32),
            pltpu.SemaphoreType.DMA((NSEM,)),
        ],
        compiler_params=pltpu.CompilerParams(
            dimension_semantics=("arbitrary",),
        ),
    )(pos_embed, idx2d)


def kernel(batch_size, pos_embed, positions):
    return _run(pos_embed, positions)
